# baseline (speedup 1.0000x reference)
import functools

import jax
import jax.numpy as jnp
from jax import lax
from jax.experimental import pallas as pl
from jax.experimental.pallas import tpu as pltpu
from jax.experimental.pallas import tpu_sc as plsc

N_ROWS = 20000
N_SEL = 1001
NC = 2
NS = 16
NW = NC * NS
B = 1024
BPW = B // NW
INPUT_SIZE = 640.0

_mesh = plsc.VectorSubcoreMesh(core_axis_name="c", subcore_axis_name="s")


@functools.partial(
    pl.kernel,
    mesh=_mesh,
    out_type=[
        jax.ShapeDtypeStruct((B,), jnp.float32),
        jax.ShapeDtypeStruct((B * 4,), jnp.int32),
        jax.ShapeDtypeStruct((B * 10,), jnp.int32),
    ],
    scratch_types=[
        pltpu.VMEM((BPW,), jnp.int32),
        pltpu.VMEM((BPW,), jnp.float32),
        pltpu.VMEM((128,), jnp.int32),
        pltpu.VMEM((128,), jnp.int32),
        pltpu.VMEM((128,), jnp.float32),
        pltpu.VMEM((128,), jnp.int32),
        pltpu.VMEM((128,), jnp.int32),
        pltpu.VMEM((64,), jnp.int32),
        pltpu.VMEM((128,), jnp.int32),
        pltpu.VMEM((128,), jnp.int32),
        pltpu.VMEM((64,), jnp.int32),
        pltpu.VMEM((128,), jnp.float32),
        pltpu.VMEM((128,), jnp.float32),
        pltpu.VMEM((64,), jnp.float32),
        pltpu.VMEM((16,), jnp.int32),
        pltpu.VMEM((16,), jnp.int32),
        pltpu.VMEM((16,), jnp.int32),
        pltpu.VMEM((16,), jnp.int32),
        pltpu.VMEM((128,), jnp.int32),
        pltpu.VMEM((320,), jnp.int32),
        pltpu.SemaphoreType.DMA,
        pltpu.SemaphoreType.DMA,
        pltpu.SemaphoreType.DMA,
        pltpu.SemaphoreType.DMA,
        pltpu.SemaphoreType.DMA,
        pltpu.SemaphoreType.DMA,
        pltpu.SemaphoreType.DMA,
    ],
)
def _postprocess_sc(org_hbm, idx_hbm, s_hbm, b_hbm, k_hbm, posb_hbm, posk_hbm,
                    out_s, out_b, out_k,
                    idx_v, sg, pb, rb, bg, pk0, pk1, pk2,
                    rk0, rk1, rk2, kg0, kg1, kg2,
                    obidx, oinv, bnd_i, inv_i, bout, kout,
                    sem_i, sem_rb, sem_rk, sem_o, sem_s, sem_b, sem_k):
    wid = lax.axis_index("s") * NC + lax.axis_index("c")
    base = wid * BPW
    lane = lax.iota(jnp.int32, 16)

    obidx[pl.ds(0, 16)] = (lane + 1) & 1
    oinv[pl.ds(0, 16)] = lane & 1
    c_o1 = pltpu.async_copy(org_hbm.at[obidx], bnd_i, sem_o)
    c_o2 = pltpu.async_copy(org_hbm.at[oinv], inv_i, sem_o)

    c_i = pltpu.async_copy(idx_hbm.at[pl.ds(base, BPW)], idx_v, sem_i)
    c_pb = pltpu.async_copy(posb_hbm.at[pl.ds(base * 4, 128)], pb, sem_i)
    c_pk0 = pltpu.async_copy(posk_hbm.at[pl.ds(base * 10, 128)], pk0, sem_i)
    c_pk1 = pltpu.async_copy(
        posk_hbm.at[pl.ds(base * 10 + 128, 128)], pk1, sem_i)
    c_pk2 = pltpu.async_copy(
        posk_hbm.at[pl.ds(base * 10 + 256, 64)], pk2, sem_i)
    c_i.wait()
    c_pb.wait()
    c_pk0.wait()
    c_pk1.wait()
    c_pk2.wait()

    c_s = pltpu.async_copy(s_hbm.at[idx_v], sg, sem_s)
    c_rb = pltpu.async_copy(idx_hbm.at[pb], rb, sem_rb)
    c_rk0 = pltpu.async_copy(idx_hbm.at[pk0], rk0, sem_rk)
    c_rk1 = pltpu.async_copy(idx_hbm.at[pk1], rk1, sem_rk)
    c_rk2 = pltpu.async_copy(idx_hbm.at[pk2], rk2, sem_rk)

    c_rb.wait()
    for v in range(8):
        ii = lane + v * 16
        pb[pl.ds(v * 16, 16)] = rb[pl.ds(v * 16, 16)] * 4 + ((ii & 3) ^ 1)
    c_b = pltpu.async_copy(b_hbm.at[pb], bg, sem_b)

    c_rk0.wait()
    c_rk1.wait()
    c_rk2.wait()
    for v in range(20):
        ii = lane + v * 16
        if v < 8:
            pk0[pl.ds(v * 16, 16)] = rk0[pl.ds(v * 16, 16)] * 10 + ii % 10
        elif v < 16:
            pk1[pl.ds((v - 8) * 16, 16)] = (
                rk1[pl.ds((v - 8) * 16, 16)] * 10 + ii % 10)
        else:
            pk2[pl.ds((v - 16) * 16, 16)] = (
                rk2[pl.ds((v - 16) * 16, 16)] * 10 + ii % 10)
    c_k0 = pltpu.async_copy(k_hbm.at[pk0], kg0, sem_k)
    c_k1 = pltpu.async_copy(k_hbm.at[pk1], kg1, sem_k)
    c_k2 = pltpu.async_copy(k_hbm.at[pk2], kg2, sem_k)

    c_o1.wait()
    c_o2.wait()
    bound = bnd_i[...].astype(jnp.float32)
    m_vec = jnp.maximum(bound, inv_i[...].astype(jnp.float32))

    c_b.wait()
    for v in range(8):
        t = bg[pl.ds(v * 16, 16)] * m_vec / INPUT_SIZE
        bout[pl.ds(v * 16, 16)] = jnp.clip(t, 0.0, bound).astype(jnp.int32)

    c_k0.wait()
    c_k1.wait()
    c_k2.wait()
    for v in range(20):
        if v < 8:
            g = kg0[pl.ds(v * 16, 16)]
        elif v < 16:
            g = kg1[pl.ds((v - 8) * 16, 16)]
        else:
            g = kg2[pl.ds((v - 16) * 16, 16)]
        t = g * m_vec / INPUT_SIZE
        kout[pl.ds(v * 16, 16)] = jnp.clip(t, 0.0, bound).astype(jnp.int32)

    c_s.wait()
    pltpu.sync_copy(sg, out_s.at[pl.ds(base, BPW)])
    pltpu.sync_copy(bout, out_b.at[pl.ds(base * 4, 128)])
    pltpu.sync_copy(kout, out_k.at[pl.ds(base * 10, 320)])


def kernel(org_size, scores, bboxes, kpss, selected_idx):
    idx = jnp.concatenate([
        jnp.zeros((1,), jnp.int32),
        selected_idx[:, 2].astype(jnp.int32),
        jnp.zeros((B - N_SEL,), jnp.int32),
    ])
    org = jnp.concatenate(
        [org_size.astype(jnp.int32), jnp.zeros((14,), jnp.int32)])
    pos_b = jnp.arange(B * 4, dtype=jnp.int32) // 4
    pos_k = jnp.arange(B * 10, dtype=jnp.int32) // 10
    out_s, out_b, out_k = _postprocess_sc(
        org, idx,
        scores.reshape(N_ROWS),
        bboxes.reshape(N_ROWS * 4),
        kpss.reshape(N_ROWS * 10),
        pos_b, pos_k,
    )
    return (out_s[:N_SEL],
            out_b.reshape(B, 4)[:N_SEL],
            out_k.reshape(B, 5, 2)[:N_SEL])


# speedup vs baseline: 1.0236x; 1.0236x over previous
import functools

import jax
import jax.numpy as jnp
import numpy as np
from jax import lax
from jax.experimental import pallas as pl
from jax.experimental.pallas import tpu as pltpu
from jax.experimental.pallas import tpu_sc as plsc

N_ROWS = 20000
N_SEL = 1001
NC = 2
NS = 16
NW = NC * NS
B = 1024
BPW = B // NW
PPW = BPW + BPW * 4 + BPW * 10
INPUT_SIZE = 640.0

_mesh = plsc.VectorSubcoreMesh(core_axis_name="c", subcore_axis_name="s")


def _build_positions() -> np.ndarray:
    rows = np.arange(B, dtype=np.int64)
    pos_row = np.clip(3 * rows - 1, 0, 3 * (N_SEL - 1) - 1)
    chunks = []
    for w in range(NW):
        r = pos_row[w * BPW:(w + 1) * BPW]
        chunks.append(r)
        chunks.append(np.repeat(r, 4))
        chunks.append(np.repeat(r, 10))
    return np.concatenate(chunks).astype(np.int32)


_POSITIONS = _build_positions()
_ORG_PAT = np.concatenate([
    (np.arange(16) + 1) % 2, np.arange(16) % 2]).astype(np.int32)


@functools.partial(
    pl.kernel,
    mesh=_mesh,
    out_type=[
        jax.ShapeDtypeStruct((B,), jnp.float32),
        jax.ShapeDtypeStruct((B * 4,), jnp.int32),
        jax.ShapeDtypeStruct((B * 10,), jnp.int32),
    ],
    scratch_types=[
        pltpu.VMEM((PPW,), jnp.int32),
        pltpu.VMEM((32,), jnp.int32),
        pltpu.VMEM((32,), jnp.int32),
        pltpu.VMEM((BPW,), jnp.int32),
        pltpu.VMEM((BPW,), jnp.float32),
        pltpu.VMEM((128,), jnp.int32),
        pltpu.VMEM((128,), jnp.int32),
        pltpu.VMEM((128,), jnp.float32),
        pltpu.VMEM((128,), jnp.int32),
        pltpu.VMEM((128,), jnp.int32),
        pltpu.VMEM((64,), jnp.int32),
        pltpu.VMEM((128,), jnp.int32),
        pltpu.VMEM((128,), jnp.int32),
        pltpu.VMEM((64,), jnp.int32),
        pltpu.VMEM((128,), jnp.float32),
        pltpu.VMEM((128,), jnp.float32),
        pltpu.VMEM((64,), jnp.float32),
        pltpu.VMEM((128,), jnp.int32),
        pltpu.VMEM((320,), jnp.int32),
        pltpu.SemaphoreType.DMA,
        pltpu.SemaphoreType.DMA,
        pltpu.SemaphoreType.DMA,
        pltpu.SemaphoreType.DMA,
        pltpu.SemaphoreType.DMA,
        pltpu.SemaphoreType.DMA,
    ],
)
def _postprocess_sc(org_hbm, sel_hbm, s_hbm, b_hbm, k_hbm, pos_hbm, opat_hbm,
                    out_s, out_b, out_k,
                    pos_v, opat_v, org_g, idx_v, sg, rb, eb, bg,
                    rk0, rk1, rk2, ek0, ek1, ek2, kg0, kg1, kg2, bout, kout,
                    sem_p, sem_r1, sem_o, sem_s, sem_b, sem_k):
    wid = lax.axis_index("s") * NC + lax.axis_index("c")
    lane = lax.iota(jnp.int32, 16)

    c_p = pltpu.async_copy(pos_hbm.at[pl.ds(wid * PPW, PPW)], pos_v, sem_p)
    c_pat = pltpu.async_copy(opat_hbm, opat_v, sem_p)
    c_p.wait()
    c_pat.wait()

    c_o = pltpu.async_copy(org_hbm.at[opat_v], org_g, sem_o)
    c_rs = pltpu.async_copy(sel_hbm.at[pos_v.at[pl.ds(0, 32)]], idx_v, sem_r1)
    c_rb = pltpu.async_copy(sel_hbm.at[pos_v.at[pl.ds(32, 128)]], rb, sem_r1)
    c_rk0 = pltpu.async_copy(
        sel_hbm.at[pos_v.at[pl.ds(160, 128)]], rk0, sem_r1)
    c_rk1 = pltpu.async_copy(
        sel_hbm.at[pos_v.at[pl.ds(288, 128)]], rk1, sem_r1)
    c_rk2 = pltpu.async_copy(
        sel_hbm.at[pos_v.at[pl.ds(416, 64)]], rk2, sem_r1)
    c_rs.wait()
    c_rb.wait()
    c_rk0.wait()
    c_rk1.wait()
    c_rk2.wait()

    @pl.when(wid == 0)
    def _patch():
        idx_v[pl.ds(0, 16)] = jnp.where(lane < 1, 0, idx_v[pl.ds(0, 16)])
        rb[pl.ds(0, 16)] = jnp.where(lane < 4, 0, rb[pl.ds(0, 16)])
        rk0[pl.ds(0, 16)] = jnp.where(lane < 10, 0, rk0[pl.ds(0, 16)])

    c_s = pltpu.async_copy(s_hbm.at[idx_v], sg, sem_s)
    for v in range(8):
        ii = lane + v * 16
        eb[pl.ds(v * 16, 16)] = rb[pl.ds(v * 16, 16)] * 4 + ((ii & 3) ^ 1)
    c_b = pltpu.async_copy(b_hbm.at[eb], bg, sem_b)
    for v in range(20):
        ii = lane + v * 16
        if v < 8:
            ek0[pl.ds(v * 16, 16)] = rk0[pl.ds(v * 16, 16)] * 10 + ii % 10
        elif v < 16:
            ek1[pl.ds((v - 8) * 16, 16)] = (
                rk1[pl.ds((v - 8) * 16, 16)] * 10 + ii % 10)
        else:
            ek2[pl.ds((v - 16) * 16, 16)] = (
                rk2[pl.ds((v - 16) * 16, 16)] * 10 + ii % 10)
    c_k0 = pltpu.async_copy(k_hbm.at[ek0], kg0, sem_k)
    c_k1 = pltpu.async_copy(k_hbm.at[ek1], kg1, sem_k)
    c_k2 = pltpu.async_copy(k_hbm.at[ek2], kg2, sem_k)

    c_o.wait()
    bound = org_g[pl.ds(0, 16)].astype(jnp.float32)
    m_vec = jnp.maximum(bound, org_g[pl.ds(16, 16)].astype(jnp.float32))

    c_b.wait()
    for v in range(8):
        t = bg[pl.ds(v * 16, 16)] * m_vec / INPUT_SIZE
        bout[pl.ds(v * 16, 16)] = jnp.clip(t, 0.0, bound).astype(jnp.int32)

    c_k0.wait()
    c_k1.wait()
    c_k2.wait()
    for v in range(20):
        if v < 8:
            g = kg0[pl.ds(v * 16, 16)]
        elif v < 16:
            g = kg1[pl.ds((v - 8) * 16, 16)]
        else:
            g = kg2[pl.ds((v - 16) * 16, 16)]
        t = g * m_vec / INPUT_SIZE
        kout[pl.ds(v * 16, 16)] = jnp.clip(t, 0.0, bound).astype(jnp.int32)

    base = wid * BPW
    c_s.wait()
    pltpu.sync_copy(sg, out_s.at[pl.ds(base, BPW)])
    pltpu.sync_copy(bout, out_b.at[pl.ds(base * 4, 128)])
    pltpu.sync_copy(kout, out_k.at[pl.ds(base * 10, 320)])


def kernel(org_size, scores, bboxes, kpss, selected_idx):
    out_s, out_b, out_k = _postprocess_sc(
        org_size.astype(jnp.int32),
        selected_idx.astype(jnp.int32).reshape(3 * (N_SEL - 1)),
        scores.reshape(N_ROWS),
        bboxes.reshape(N_ROWS * 4),
        kpss.reshape(N_ROWS * 10),
        jnp.asarray(_POSITIONS),
        jnp.asarray(_ORG_PAT),
    )
    return (out_s[:N_SEL],
            out_b.reshape(B, 4)[:N_SEL],
            out_k.reshape(B, 5, 2)[:N_SEL])


# speedup vs baseline: 2.9968x; 2.9277x over previous
import functools

import jax
import jax.numpy as jnp
import numpy as np
from jax import lax
from jax.experimental import pallas as pl
from jax.experimental.pallas import tpu as pltpu
from jax.experimental.pallas import tpu_sc as plsc

N_ROWS = 20000
N_SEL = 1001
NC = 2
NS = 16
NW = NC * NS
B = 1024
BPW = B // NW
PPW = BPW + BPW * 4 + BPW * 10
INPUT_SIZE = 640.0

_mesh = plsc.VectorSubcoreMesh(core_axis_name="c", subcore_axis_name="s")


def _build_positions() -> np.ndarray:
    rows = np.arange(B, dtype=np.int64)
    pos_row = np.clip(1999 + rows, 0, 2999)
    chunks = []
    for w in range(NW):
        r = pos_row[w * BPW:(w + 1) * BPW]
        chunks.append(r)
        chunks.append(np.repeat(r, 4))
        chunks.append(np.repeat(r, 10))
    return np.concatenate(chunks).astype(np.int32)


_POSITIONS = _build_positions()
_ORG_PAT = np.concatenate([
    (np.arange(16) + 1) % 2, np.arange(16) % 2]).astype(np.int32)


@functools.partial(
    pl.kernel,
    mesh=_mesh,
    out_type=[
        jax.ShapeDtypeStruct((B,), jnp.float32),
        jax.ShapeDtypeStruct((B * 4,), jnp.int32),
        jax.ShapeDtypeStruct((B * 10,), jnp.int32),
    ],
    scratch_types=[
        pltpu.VMEM((PPW,), jnp.int32),
        pltpu.VMEM((32,), jnp.int32),
        pltpu.VMEM((32,), jnp.int32),
        pltpu.VMEM((BPW,), jnp.int32),
        pltpu.VMEM((BPW,), jnp.float32),
        pltpu.VMEM((128,), jnp.int32),
        pltpu.VMEM((128,), jnp.int32),
        pltpu.VMEM((128,), jnp.float32),
        pltpu.VMEM((128,), jnp.int32),
        pltpu.VMEM((128,), jnp.int32),
        pltpu.VMEM((64,), jnp.int32),
        pltpu.VMEM((128,), jnp.int32),
        pltpu.VMEM((128,), jnp.int32),
        pltpu.VMEM((64,), jnp.int32),
        pltpu.VMEM((128,), jnp.float32),
        pltpu.VMEM((128,), jnp.float32),
        pltpu.VMEM((64,), jnp.float32),
        pltpu.VMEM((128,), jnp.int32),
        pltpu.VMEM((320,), jnp.int32),
        pltpu.SemaphoreType.DMA,
        pltpu.SemaphoreType.DMA,
        pltpu.SemaphoreType.DMA,
        pltpu.SemaphoreType.DMA,
        pltpu.SemaphoreType.DMA,
        pltpu.SemaphoreType.DMA,
    ],
)
def _postprocess_sc(org_hbm, sel_hbm, s_hbm, b_hbm, k_hbm, pos_hbm, opat_hbm,
                    out_s, out_b, out_k,
                    pos_v, opat_v, org_g, idx_v, sg, rb, eb, bg,
                    rk0, rk1, rk2, ek0, ek1, ek2, kg0, kg1, kg2, bout, kout,
                    sem_p, sem_r1, sem_o, sem_s, sem_b, sem_k):
    wid = lax.axis_index("s") * NC + lax.axis_index("c")
    lane = lax.iota(jnp.int32, 16)

    c_p = pltpu.async_copy(pos_hbm.at[pl.ds(wid * PPW, PPW)], pos_v, sem_p)
    c_pat = pltpu.async_copy(opat_hbm, opat_v, sem_p)
    c_p.wait()
    c_pat.wait()

    c_o = pltpu.async_copy(org_hbm.at[opat_v], org_g, sem_o)
    c_rs = pltpu.async_copy(sel_hbm.at[pos_v.at[pl.ds(0, 32)]], idx_v, sem_r1)
    c_rb = pltpu.async_copy(sel_hbm.at[pos_v.at[pl.ds(32, 128)]], rb, sem_r1)
    c_rk0 = pltpu.async_copy(
        sel_hbm.at[pos_v.at[pl.ds(160, 128)]], rk0, sem_r1)
    c_rk1 = pltpu.async_copy(
        sel_hbm.at[pos_v.at[pl.ds(288, 128)]], rk1, sem_r1)
    c_rk2 = pltpu.async_copy(
        sel_hbm.at[pos_v.at[pl.ds(416, 64)]], rk2, sem_r1)
    c_rs.wait()
    c_rb.wait()
    c_rk0.wait()
    c_rk1.wait()
    c_rk2.wait()

    @pl.when(wid == 0)
    def _patch():
        idx_v[pl.ds(0, 16)] = jnp.where(lane < 1, 0, idx_v[pl.ds(0, 16)])
        rb[pl.ds(0, 16)] = jnp.where(lane < 4, 0, rb[pl.ds(0, 16)])
        rk0[pl.ds(0, 16)] = jnp.where(lane < 10, 0, rk0[pl.ds(0, 16)])

    c_s = pltpu.async_copy(s_hbm.at[idx_v], sg, sem_s)
    bcol = ((lane & 3) ^ 1) * N_ROWS
    for v in range(8):
        eb[pl.ds(v * 16, 16)] = rb[pl.ds(v * 16, 16)] + bcol
    c_b = pltpu.async_copy(b_hbm.at[eb], bg, sem_b)
    for v in range(20):
        kcol = ((lane + v * 16) % 10) * N_ROWS
        if v < 8:
            ek0[pl.ds(v * 16, 16)] = rk0[pl.ds(v * 16, 16)] + kcol
        elif v < 16:
            ek1[pl.ds((v - 8) * 16, 16)] = rk1[pl.ds((v - 8) * 16, 16)] + kcol
        else:
            ek2[pl.ds((v - 16) * 16, 16)] = (
                rk2[pl.ds((v - 16) * 16, 16)] + kcol)
    c_k0 = pltpu.async_copy(k_hbm.at[ek0], kg0, sem_k)
    c_k1 = pltpu.async_copy(k_hbm.at[ek1], kg1, sem_k)
    c_k2 = pltpu.async_copy(k_hbm.at[ek2], kg2, sem_k)

    c_o.wait()
    bound = org_g[pl.ds(0, 16)].astype(jnp.float32)
    m_vec = jnp.maximum(bound, org_g[pl.ds(16, 16)].astype(jnp.float32))

    c_b.wait()
    for v in range(8):
        t = bg[pl.ds(v * 16, 16)] * m_vec / INPUT_SIZE
        bout[pl.ds(v * 16, 16)] = jnp.clip(t, 0.0, bound).astype(jnp.int32)

    c_k0.wait()
    c_k1.wait()
    c_k2.wait()
    for v in range(20):
        if v < 8:
            g = kg0[pl.ds(v * 16, 16)]
        elif v < 16:
            g = kg1[pl.ds((v - 8) * 16, 16)]
        else:
            g = kg2[pl.ds((v - 16) * 16, 16)]
        t = g * m_vec / INPUT_SIZE
        kout[pl.ds(v * 16, 16)] = jnp.clip(t, 0.0, bound).astype(jnp.int32)

    base = wid * BPW
    c_s.wait()
    pltpu.sync_copy(sg, out_s.at[pl.ds(base, BPW)])
    pltpu.sync_copy(bout, out_b.at[pl.ds(base * 4, 128)])
    pltpu.sync_copy(kout, out_k.at[pl.ds(base * 10, 320)])


def kernel(org_size, scores, bboxes, kpss, selected_idx):
    sel_t = jnp.transpose(selected_idx.astype(jnp.int32)).reshape(3000)
    bb_t = jnp.transpose(bboxes, (0, 2, 1)).reshape(4 * N_ROWS)
    kp_t = jnp.transpose(kpss, (1, 2, 0)).reshape(10 * N_ROWS)
    out_s, out_b, out_k = _postprocess_sc(
        org_size.astype(jnp.int32),
        sel_t,
        scores.reshape(N_ROWS),
        bb_t,
        kp_t,
        jnp.asarray(_POSITIONS),
        jnp.asarray(_ORG_PAT),
    )
    return (out_s[:N_SEL],
            out_b.reshape(B, 4)[:N_SEL],
            out_k.reshape(B, 5, 2)[:N_SEL])


# speedup vs baseline: 4.3634x; 1.4560x over previous
import functools

import jax
import jax.numpy as jnp
from jax import lax
from jax.experimental import pallas as pl
from jax.experimental.pallas import tpu as pltpu
from jax.experimental.pallas import tpu_sc as plsc

N_ROWS = 20000
N_SEL = 1001
NC = 2
NS = 16
NW = NC * NS
B = 1024
BPW = B // NW
INPUT_SIZE = 640.0

_mesh = plsc.VectorSubcoreMesh(core_axis_name="c", subcore_axis_name="s")


@functools.partial(
    pl.kernel,
    mesh=_mesh,
    out_type=[
        jax.ShapeDtypeStruct((B,), jnp.float32),
        jax.ShapeDtypeStruct((B * 4,), jnp.int32),
        jax.ShapeDtypeStruct((B * 10,), jnp.int32),
    ],
    scratch_types=[
        pltpu.VMEM((BPW,), jnp.int32),
        pltpu.VMEM((32,), jnp.int32),
        pltpu.VMEM((32,), jnp.int32),
        pltpu.VMEM((BPW,), jnp.int32),
        pltpu.VMEM((BPW,), jnp.float32),
        pltpu.VMEM((128,), jnp.int32),
        pltpu.VMEM((128,), jnp.float32),
        pltpu.VMEM((320,), jnp.int32),
        pltpu.VMEM((320,), jnp.float32),
        pltpu.VMEM((128,), jnp.int32),
        pltpu.VMEM((320,), jnp.int32),
        pltpu.SemaphoreType.DMA,
        pltpu.SemaphoreType.DMA,
        pltpu.SemaphoreType.DMA,
        pltpu.SemaphoreType.DMA,
        pltpu.SemaphoreType.DMA,
    ],
)
def _postprocess_sc(org_hbm, sel_hbm, s_hbm, b_hbm, k_hbm,
                    out_s, out_b, out_k,
                    pos_v, opat_v, org_g, idx_v, sg, eb, bg, ek, kg,
                    bout, kout,
                    sem_r, sem_o, sem_s, sem_b, sem_k):
    wid = lax.axis_index("s") * NC + lax.axis_index("c")
    lane = lax.iota(jnp.int32, 16)

    for k in range(NW):
        @pl.when(wid == k)
        def _store_pos(k=k):
            pos_v[pl.ds(0, 16)] = jnp.minimum(lane + (1999 + 32 * k), 2999)
            pos_v[pl.ds(16, 16)] = jnp.minimum(lane + (2015 + 32 * k), 2999)
    opat_v[pl.ds(0, 16)] = lane * 0 + 1
    opat_v[pl.ds(16, 16)] = lane * 0

    c_o = pltpu.async_copy(org_hbm.at[opat_v], org_g, sem_o)
    c_r = pltpu.async_copy(sel_hbm.at[pos_v], idx_v, sem_r)
    c_r.wait()

    @pl.when(wid == 0)
    def _patch():
        idx_v[pl.ds(0, 16)] = jnp.where(lane < 1, 0, idx_v[pl.ds(0, 16)])

    c_s = pltpu.async_copy(s_hbm.at[idx_v], sg, sem_s)
    iv0 = idx_v[pl.ds(0, 16)]
    iv1 = idx_v[pl.ds(16, 16)]
    for c in range(4):
        col = (c ^ 1) * N_ROWS
        eb[pl.ds(32 * c, 16)] = iv0 + col
        eb[pl.ds(32 * c + 16, 16)] = iv1 + col
    cbs = [
        pltpu.async_copy(
            b_hbm.at[eb.at[pl.ds(32 * c, 32)]], bg.at[pl.ds(32 * c, 32)],
            sem_b)
        for c in range(4)
    ]
    for m in range(10):
        ek[pl.ds(32 * m, 16)] = iv0 + m * N_ROWS
        ek[pl.ds(32 * m + 16, 16)] = iv1 + m * N_ROWS
    cks = [
        pltpu.async_copy(
            k_hbm.at[ek.at[pl.ds(32 * m, 32)]], kg.at[pl.ds(32 * m, 32)],
            sem_k)
        for m in range(10)
    ]

    c_o.wait()
    w_vec = org_g[pl.ds(0, 16)].astype(jnp.float32)
    h_vec = org_g[pl.ds(16, 16)].astype(jnp.float32)
    m_vec = jnp.maximum(w_vec, h_vec)

    tile = wid >> 2
    rlo = (wid & 3) * 32

    for c_ in cbs:
        c_.wait()
    for c in range(4):
        bound = w_vec if c % 2 == 0 else h_vec
        for half in range(2):
            g = bg[pl.ds(32 * c + 16 * half, 16)]
            t = g * m_vec / INPUT_SIZE
            bout[pl.ds(32 * c + 16 * half, 16)] = (
                jnp.clip(t, 0.0, bound).astype(jnp.int32))
        pltpu.sync_copy(
            bout.at[pl.ds(32 * c, 32)],
            out_b.at[pl.ds(tile * 512 + c * 128 + rlo, 32)])

    for c_ in cks:
        c_.wait()
    for m in range(10):
        bound = w_vec if m % 2 == 0 else h_vec
        for half in range(2):
            g = kg[pl.ds(32 * m + 16 * half, 16)]
            t = g * m_vec / INPUT_SIZE
            kout[pl.ds(32 * m + 16 * half, 16)] = (
                jnp.clip(t, 0.0, bound).astype(jnp.int32))
        pltpu.sync_copy(
            kout.at[pl.ds(32 * m, 32)],
            out_k.at[pl.ds((m >> 1) * 2048 + (m & 1) * 128
                           + tile * 256 + rlo, 32)])

    c_s.wait()
    pltpu.sync_copy(sg, out_s.at[pl.ds(wid * BPW, BPW)])


def kernel(org_size, scores, bboxes, kpss, selected_idx):
    sel_t = jnp.transpose(selected_idx.astype(jnp.int32)).reshape(3000)
    bb_t = jnp.transpose(bboxes, (0, 2, 1)).reshape(4 * N_ROWS)
    kp_t = jnp.transpose(kpss, (1, 2, 0)).reshape(10 * N_ROWS)
    out_s, out_b, out_k = _postprocess_sc(
        org_size.astype(jnp.int32),
        sel_t,
        scores.reshape(N_ROWS),
        bb_t,
        kp_t,
    )
    bb_o = out_b.reshape(8, 4, 128).transpose(0, 2, 1).reshape(B, 4)
    kp_o = out_k.reshape(5, 8, 2, 128).transpose(1, 3, 0, 2).reshape(B, 5, 2)
    return (out_s[:N_SEL], bb_o[:N_SEL], kp_o[:N_SEL])
